# baseline (speedup 1.0000x reference)
import functools

import jax
import jax.numpy as jnp
from jax import lax
from jax.experimental import pallas as pl
from jax.experimental.pallas import tpu as pltpu
from jax.experimental.pallas import tpu_sc as plsc


@functools.partial(jax.jit, static_argnums=(5, 6, 7, 8))
def _sc_gather_concat(emb_t, emb_c, idx, top, bottom, HW, NC, NS, B):
    H, W = HW
    D = H * W
    NW = NC * NS
    b_per_w = B // NW
    LANES = 16
    W_VECS = W // LANES
    NBUF = 2
    CHUNK = H // NBUF

    mesh = plsc.VectorSubcoreMesh(core_axis_name="c", subcore_axis_name="s")

    @functools.partial(
        pl.kernel,
        out_type=jax.ShapeDtypeStruct((B, 4, H, W), jnp.float32),
        mesh=mesh,
        scratch_types=[
            pltpu.VMEM((16, 8), jnp.int32),
            pltpu.VMEM((1, D), jnp.float32),
            pltpu.VMEM((CHUNK, W), jnp.float32),
            pltpu.VMEM((CHUNK, W), jnp.float32),
            pltpu.SemaphoreType.DMA,
            pltpu.SemaphoreType.DMA,
            pltpu.SemaphoreType.DMA,
            pltpu.SemaphoreType.DMA,
            pltpu.SemaphoreType.DMA,
            pltpu.SemaphoreType.DMA,
        ],
    )
    def sc_fn(emb_t_r, emb_c_r, idx_r, top_r, bot_r, out_r,
              idx_v, bufrow, hb0, hb1,
              sg0, sg1, si0, si1, so0, so1):
        wid = lax.axis_index("s") * NC + lax.axis_index("c")
        b0 = wid * b_per_w
        pltpu.sync_copy(idx_r.at[wid], idx_v)

        hbs = (hb0, hb1)
        in_sems = (si0, si1)
        out_sems = (so0, so1)

        def permute_chunk(hb, h):
            @plsc.parallel_loop(0, CHUNK, 1, unroll=8)
            def _(r):
                base = pl.multiple_of((h * CHUNK + r) * W, LANES)
                for c in range(W_VECS):
                    hb[r, pl.ds(c * LANES, LANES)] = (
                        bufrow[0, pl.ds(base + c * LANES, LANES)])

        gathers = []
        tasks = []
        for jj in range(b_per_w):
            gathers.append((emb_t_r, jj))
            gathers.append((emb_c_r, b_per_w + jj))
            tasks.append(("d", top_r, jj, 2))
            tasks.append(("g", None, jj, 0))
            tasks.append(("d", bot_r, jj, 3))
            tasks.append(("g", None, jj, 1))

        HALF_D = CHUNK * W
        gather_sems = (sg0, sg1)

        def start_gather(i):
            src, p = gathers[i]
            return [
                pltpu.async_copy(
                    src.at[idx_v.at[p, pl.ds(0, 1)], pl.ds(h * HALF_D, HALF_D)],
                    bufrow.at[:, pl.ds(h * HALF_D, HALF_D)], gather_sems[h])
                for h in range(NBUF)
            ]

        state = {"slot": 0, "out": [None] * NBUF}

        def emit_chunk(kind, src, b, ch, h):
            s = state["slot"]
            state["slot"] = (s + 1) % NBUF
            if state["out"][s] is not None:
                state["out"][s].wait()
            if kind == "d":
                pltpu.async_copy(
                    src.at[b, 0, pl.ds(h * CHUNK, CHUNK)], hbs[s], in_sems[s]
                ).wait()
            else:
                permute_chunk(hbs[s], h)
            state["out"][s] = pltpu.async_copy(
                hbs[s], out_r.at[b, ch, pl.ds(h * CHUNK, CHUNK)], out_sems[s])

        gi = 0
        gh = start_gather(0)
        gi = 1
        for kind, src, jj, ch in tasks:
            b = b0 + jj
            if kind == "g":
                for h in range(NBUF):
                    gh[h].wait()
                    emit_chunk("g", None, b, ch, h)
                if gi < len(gathers):
                    gh = start_gather(gi)
                    gi += 1
            else:
                for h in range(NBUF):
                    emit_chunk("d", src, b, ch, h)
        for s in range(NBUF):
            if state["out"][s] is not None:
                state["out"][s].wait()

    return sc_fn(emb_t, emb_c, idx, top, bottom)


def kernel(x, t, c_idx, top, bottom, emb_t, emb_c):
    B = x.shape[0]
    H = x.shape[2]
    W = x.shape[3]

    info = plsc.get_sparse_core_info()
    NC, NS = info.num_cores, info.num_subcores
    NW = NC * NS
    b_per_w = B // NW

    t_i = t.astype(jnp.int32).reshape(NW, b_per_w)
    c_i = c_idx.astype(jnp.int32).reshape(NW, b_per_w)
    pad = jnp.zeros((NW, 16 - 2 * b_per_w), jnp.int32)
    vals = jnp.concatenate([t_i, c_i, pad], axis=1)
    idx = jnp.broadcast_to(vals[:, :, None], (NW, 16, 8))

    return _sc_gather_concat(emb_t, emb_c, idx, top, bottom, (H, W), NC, NS, B)


# speedup vs baseline: 1.0059x; 1.0059x over previous
import functools

import jax
import jax.numpy as jnp
from jax import lax
from jax.experimental import pallas as pl
from jax.experimental.pallas import tpu as pltpu
from jax.experimental.pallas import tpu_sc as plsc


@functools.partial(jax.jit, static_argnums=(5, 6, 7, 8))
def _sc_gather_concat(emb_t, emb_c, idx, top, bottom, HW, NC, NS, B):
    H, W = HW
    D = H * W
    NW = NC * NS
    b_per_w = B // NW
    LANES = 16
    W_VECS = W // LANES
    NBUF = 2
    CHUNK = H // NBUF

    mesh = plsc.VectorSubcoreMesh(core_axis_name="c", subcore_axis_name="s")

    @functools.partial(
        pl.kernel,
        out_type=jax.ShapeDtypeStruct((B, 4, H, W), jnp.float32),
        mesh=mesh,
        scratch_types=[
            pltpu.VMEM((16, 8), jnp.int32),
            pltpu.VMEM((1, D), jnp.float32),
            pltpu.VMEM((CHUNK, W), jnp.float32),
            pltpu.VMEM((CHUNK, W), jnp.float32),
            pltpu.SemaphoreType.DMA,
            pltpu.SemaphoreType.DMA,
            pltpu.SemaphoreType.DMA,
            pltpu.SemaphoreType.DMA,
            pltpu.SemaphoreType.DMA,
            pltpu.SemaphoreType.DMA,
        ],
    )
    def sc_fn(emb_t_r, emb_c_r, idx_r, top_r, bot_r, out_r,
              idx_v, bufrow, hb0, hb1,
              sg0, sg1, si0, si1, so0, so1):
        wid = lax.axis_index("s") * NC + lax.axis_index("c")
        b0 = wid * b_per_w
        pltpu.sync_copy(idx_r.at[wid], idx_v)

        hbs = (hb0, hb1)
        in_sems = (si0, si1)
        out_sems = (so0, so1)

        def permute_chunk(hb, h):
            @plsc.parallel_loop(0, CHUNK, 1, unroll=8)
            def _(r):
                base = pl.multiple_of((h * CHUNK + r) * W, LANES)
                for c in range(W_VECS):
                    hb[r, pl.ds(c * LANES, LANES)] = (
                        bufrow[0, pl.ds(base + c * LANES, LANES)])

        gathers = []
        tasks = []
        for jj in range(b_per_w):
            gathers.append((emb_t_r, jj))
            gathers.append((emb_c_r, b_per_w + jj))
            tasks.append(("d", top_r, jj, 2))
            tasks.append(("g", None, jj, 0))
            tasks.append(("d", bot_r, jj, 3))
            tasks.append(("g", None, jj, 1))

        def start_gather(i):
            src, p = gathers[i]
            return pltpu.async_copy(
                src.at[idx_v.at[p, pl.ds(0, 1)]], bufrow, sg0)

        state = {"slot": 0, "out": [None] * NBUF}

        def emit_chunk(kind, src, b, ch, h):
            s = state["slot"]
            state["slot"] = (s + 1) % NBUF
            if state["out"][s] is not None:
                state["out"][s].wait()
            if kind == "d":
                pltpu.async_copy(
                    src.at[b, 0, pl.ds(h * CHUNK, CHUNK)], hbs[s], in_sems[s]
                ).wait()
            else:
                permute_chunk(hbs[s], h)
            state["out"][s] = pltpu.async_copy(
                hbs[s], out_r.at[b, ch, pl.ds(h * CHUNK, CHUNK)], out_sems[s])

        gi = 0
        gh = start_gather(0)
        gi = 1
        for kind, src, jj, ch in tasks:
            b = b0 + jj
            if kind == "g":
                gh.wait()
                for h in range(NBUF):
                    emit_chunk("g", None, b, ch, h)
                if gi < len(gathers):
                    gh = start_gather(gi)
                    gi += 1
            else:
                for h in range(NBUF):
                    emit_chunk("d", src, b, ch, h)
        for s in range(NBUF):
            if state["out"][s] is not None:
                state["out"][s].wait()

    return sc_fn(emb_t, emb_c, idx, top, bottom)


def kernel(x, t, c_idx, top, bottom, emb_t, emb_c):
    B = x.shape[0]
    H = x.shape[2]
    W = x.shape[3]

    info = plsc.get_sparse_core_info()
    NC, NS = info.num_cores, info.num_subcores
    NW = NC * NS
    b_per_w = B // NW

    t_i = t.astype(jnp.int32).reshape(NW, b_per_w)
    c_i = c_idx.astype(jnp.int32).reshape(NW, b_per_w)
    pad = jnp.zeros((NW, 16 - 2 * b_per_w), jnp.int32)
    vals = jnp.concatenate([t_i, c_i, pad], axis=1)
    idx = jnp.broadcast_to(vals[:, :, None], (NW, 16, 8))

    return _sc_gather_concat(emb_t, emb_c, idx, top, bottom, (H, W), NC, NS, B)
